# baseline (speedup 1.0000x reference)
import functools

import jax
import jax.numpy as jnp
from jax import lax
from jax.experimental import pallas as pl
from jax.experimental.pallas import tpu as pltpu
from jax.experimental.pallas import tpu_sc as plsc

HID = 64
G4 = 4 * HID


def _sc_gather(table, idx_flat, *, chunk=320):
    V, D = table.shape
    R = idx_flat.shape[0]
    info = plsc.get_sparse_core_info()
    nw = info.num_cores * info.num_subcores
    assert R % nw == 0
    per_w = R // nw
    assert per_w % chunk == 0 and chunk % 8 == 0
    n_chunks = per_w // chunk
    mesh = plsc.VectorSubcoreMesh(core_axis_name="c", subcore_axis_name="s")

    @functools.partial(
        pl.kernel,
        mesh=mesh,
        out_type=jax.ShapeDtypeStruct((R, D), table.dtype),
        scratch_types=[
            pltpu.VMEM((chunk,), jnp.int32),
            pltpu.VMEM((chunk,), jnp.int32),
            pltpu.VMEM((chunk, D), table.dtype),
            pltpu.VMEM((chunk, D), table.dtype),
            pltpu.SemaphoreType.DMA,
            pltpu.SemaphoreType.DMA,
            pltpu.SemaphoreType.DMA,
            pltpu.SemaphoreType.DMA,
        ],
    )
    def gather_kernel(table_hbm, idx_hbm, out_hbm, idx0, idx1, rows0, rows1,
                      sg0, sg1, sw0, sw1):
        wid = lax.axis_index("s") * info.num_cores + lax.axis_index("c")
        base = wid * per_w
        idx_v = (idx0, idx1)
        rows_v = (rows0, rows1)
        sg = (sg0, sg1)
        sw = (sw0, sw1)

        def off(g):
            return base + g * chunk

        pltpu.sync_copy(idx_hbm.at[pl.ds(off(0), chunk)], idx_v[0])
        gathers = {0: pltpu.async_copy(table_hbm.at[idx_v[0]], rows_v[0], sg[0])}
        writes = {}
        for g in range(n_chunks):
            b = g & 1
            o = 1 - b
            if g + 1 < n_chunks:
                if g >= 1:
                    writes[g - 1].wait()
                pltpu.sync_copy(idx_hbm.at[pl.ds(off(g + 1), chunk)], idx_v[o])
                gathers[g + 1] = pltpu.async_copy(
                    table_hbm.at[idx_v[o]], rows_v[o], sg[o])
            gathers[g].wait()
            writes[g] = pltpu.async_copy(
                rows_v[b], out_hbm.at[pl.ds(off(g), chunk), :], sw[b])
        if n_chunks >= 2:
            writes[n_chunks - 2].wait()
        writes[n_chunks - 1].wait()

    return gather_kernel(table, idx_flat)


def _sigmoid_pre(y):
    return 0.5 * jnp.tanh(y) + 0.5


def _lstm_body(g_ref, wf_ref, uf_ref, bf_ref, wb_ref, ub_ref, bb_ref, out_ref,
               pf_ref, pb_ref):
    k, blk, d = g_ref.shape

    x_all = g_ref[...].reshape(k * blk, d).astype(jnp.bfloat16)
    dn = (((1,), (1,)), ((), ()))
    pf_ref[...] = (lax.dot_general(wf_ref[...].astype(jnp.bfloat16), x_all, dn,
                                   preferred_element_type=jnp.float32) + bf_ref[...])
    pb_ref[...] = (lax.dot_general(wb_ref[...].astype(jnp.bfloat16), x_all, dn,
                                   preferred_element_type=jnp.float32) + bb_ref[...])

    uf = uf_ref[...]
    ub = ub_ref[...]
    z = jnp.zeros((HID, blk), jnp.float32)
    h_f, c_f, h_b, c_b, acc_f, acc_b = z, z, z, z, z, z
    dm = (((1,), (0,)), ((), ()))
    for t in range(k):
        gf = (pf_ref[:, t * blk:(t + 1) * blk]
              + lax.dot_general(uf, h_f, dm, preferred_element_type=jnp.float32))
        gb = (pb_ref[:, (k - 1 - t) * blk:(k - t) * blk]
              + lax.dot_general(ub, h_b, dm, preferred_element_type=jnp.float32))
        i_f = _sigmoid_pre(gf[0:HID])
        f_f = _sigmoid_pre(gf[HID:2 * HID])
        g_f = jnp.tanh(gf[2 * HID:3 * HID])
        o_f = _sigmoid_pre(gf[3 * HID:4 * HID])
        c_f = f_f * c_f + i_f * g_f
        h_f = o_f * jnp.tanh(c_f)
        i_b = _sigmoid_pre(gb[0:HID])
        f_b = _sigmoid_pre(gb[HID:2 * HID])
        g_b = jnp.tanh(gb[2 * HID:3 * HID])
        o_b = _sigmoid_pre(gb[3 * HID:4 * HID])
        c_b = f_b * c_b + i_b * g_b
        h_b = o_b * jnp.tanh(c_b)
        acc_f = acc_f + h_f
        acc_b = acc_b + h_b
    cat = jnp.concatenate([acc_f, acc_b], axis=0) * (1.0 / k)
    out_ref[...] = jnp.transpose(cat)


def _tc_bilstm_mean(g_tmajor, wf, uf, bf, wb, ub, bb, *, blk=256):
    k, n, d = g_tmajor.shape
    assert n % blk == 0
    nb = n // blk
    return pl.pallas_call(
        _lstm_body,
        grid=(nb,),
        in_specs=[
            pl.BlockSpec((k, blk, d), lambda i: (0, i, 0)),
            pl.BlockSpec((G4, d), lambda i: (0, 0)),
            pl.BlockSpec((G4, HID), lambda i: (0, 0)),
            pl.BlockSpec((G4, 1), lambda i: (0, 0)),
            pl.BlockSpec((G4, d), lambda i: (0, 0)),
            pl.BlockSpec((G4, HID), lambda i: (0, 0)),
            pl.BlockSpec((G4, 1), lambda i: (0, 0)),
        ],
        out_specs=pl.BlockSpec((blk, 2 * HID), lambda i: (i, 0)),
        out_shape=jax.ShapeDtypeStruct((n, 2 * HID), jnp.float32),
        scratch_shapes=[
            pltpu.VMEM((G4, k * blk), jnp.float32),
            pltpu.VMEM((G4, k * blk), jnp.float32),
        ],
    )(g_tmajor, wf, uf, bf, wb, ub, bb)


def _gate_scale():
    return jnp.concatenate([
        jnp.full((2 * HID, 1), 0.5, jnp.float32),
        jnp.ones((HID, 1), jnp.float32),
        jnp.full((HID, 1), 0.5, jnp.float32),
    ], axis=0)


def _prep(wih, whh, bih, bhh, s):
    return wih * s, whh * s, (bih + bhh).reshape(G4, 1) * s


@jax.jit
def kernel(x_paper, x_author, idx_paper_to_author, idx_author_to_paper,
           p_wih_f, p_whh_f, p_bih_f, p_bhh_f, p_wih_b, p_whh_b, p_bih_b, p_bhh_b,
           a_wih_f, a_whh_f, a_bih_f, a_bhh_f, a_wih_b, a_whh_b, a_bih_b, a_bhh_b):
    n, k = idx_paper_to_author.shape
    d = x_paper.shape[1]
    blk = 256
    n_pad = (n + blk - 1) // blk * blk
    s = _gate_scale()

    def flat_idx(idx):
        idx_t = jnp.transpose(idx.astype(jnp.int32))
        idx_t = jnp.pad(idx_t, ((0, 0), (0, n_pad - n)))
        return idx_t.reshape(-1)

    pwf, puf, pbf = _prep(p_wih_f, p_whh_f, p_bih_f, p_bhh_f, s)
    pwb, pub, pbb = _prep(p_wih_b, p_whh_b, p_bih_b, p_bhh_b, s)
    awf, auf, abf = _prep(a_wih_f, a_whh_f, a_bih_f, a_bhh_f, s)
    awb, aub, abb = _prep(a_wih_b, a_whh_b, a_bih_b, a_bhh_b, s)

    g0 = _sc_gather(x_paper, flat_idx(idx_paper_to_author)).reshape(k, n_pad, d)
    out_author = _tc_bilstm_mean(g0, pwf, puf, pbf, pwb, pub, pbb, blk=blk)[:n]
    g1 = _sc_gather(x_author, flat_idx(idx_author_to_paper)).reshape(k, n_pad, d)
    out_paper = _tc_bilstm_mean(g1, awf, auf, abf, awb, aub, abb, blk=blk)[:n]
    return (out_author, out_paper)


# speedup vs baseline: 1.0028x; 1.0028x over previous
import functools

import jax
import jax.numpy as jnp
from jax import lax
from jax.experimental import pallas as pl
from jax.experimental.pallas import tpu as pltpu
from jax.experimental.pallas import tpu_sc as plsc

HID = 64
G4 = 4 * HID


def _sc_gather(table, idx_flat, *, chunk=320):
    V, D = table.shape
    R = idx_flat.shape[0]
    info = plsc.get_sparse_core_info()
    nw = info.num_cores * info.num_subcores
    assert R % nw == 0
    per_w = R // nw
    assert per_w % chunk == 0 and chunk % 8 == 0
    n_chunks = per_w // chunk
    mesh = plsc.VectorSubcoreMesh(core_axis_name="c", subcore_axis_name="s")

    @functools.partial(
        pl.kernel,
        mesh=mesh,
        out_type=jax.ShapeDtypeStruct((R, D), table.dtype),
        scratch_types=[
            pltpu.VMEM((chunk,), jnp.int32),
            pltpu.VMEM((chunk,), jnp.int32),
            pltpu.VMEM((chunk, D), table.dtype),
            pltpu.VMEM((chunk, D), table.dtype),
            pltpu.SemaphoreType.DMA,
            pltpu.SemaphoreType.DMA,
            pltpu.SemaphoreType.DMA,
            pltpu.SemaphoreType.DMA,
        ],
    )
    def gather_kernel(table_hbm, idx_hbm, out_hbm, idx0, idx1, rows0, rows1,
                      sg0, sg1, sw0, sw1):
        wid = lax.axis_index("s") * info.num_cores + lax.axis_index("c")
        base = wid * per_w
        idx_v = (idx0, idx1)
        rows_v = (rows0, rows1)
        sg = (sg0, sg1)
        sw = (sw0, sw1)

        def off(g):
            return base + g * chunk

        pltpu.sync_copy(idx_hbm.at[pl.ds(off(0), chunk)], idx_v[0])
        gathers = {0: pltpu.async_copy(table_hbm.at[idx_v[0]], rows_v[0], sg[0])}
        writes = {}
        for g in range(n_chunks):
            b = g & 1
            o = 1 - b
            if g + 1 < n_chunks:
                if g >= 1:
                    writes[g - 1].wait()
                pltpu.sync_copy(idx_hbm.at[pl.ds(off(g + 1), chunk)], idx_v[o])
                gathers[g + 1] = pltpu.async_copy(
                    table_hbm.at[idx_v[o]], rows_v[o], sg[o])
            gathers[g].wait()
            writes[g] = pltpu.async_copy(
                rows_v[b], out_hbm.at[pl.ds(off(g), chunk), :], sw[b])
        if n_chunks >= 2:
            writes[n_chunks - 2].wait()
        writes[n_chunks - 1].wait()

    return gather_kernel(table, idx_flat)


def _sigmoid_pre(y):
    return 0.5 * jnp.tanh(y) + 0.5


def _lstm_body(g_ref, wf_ref, uf_ref, bf_ref, wb_ref, ub_ref, bb_ref, out_ref,
               pf_ref, pb_ref):
    k, blk, d = g_ref.shape

    x_all = g_ref[...].reshape(k * blk, d)
    dn = (((1,), (1,)), ((), ()))
    wdt = x_all.dtype
    pf_ref[...] = (lax.dot_general(wf_ref[...].astype(wdt), x_all, dn,
                                   preferred_element_type=jnp.float32) + bf_ref[...])
    pb_ref[...] = (lax.dot_general(wb_ref[...].astype(wdt), x_all, dn,
                                   preferred_element_type=jnp.float32) + bb_ref[...])

    uf = uf_ref[...]
    ub = ub_ref[...]
    z = jnp.zeros((HID, blk), jnp.float32)
    h_f, c_f, h_b, c_b, acc_f, acc_b = z, z, z, z, z, z
    dm = (((1,), (0,)), ((), ()))
    for t in range(k):
        gf = (pf_ref[:, t * blk:(t + 1) * blk]
              + lax.dot_general(uf, h_f, dm, preferred_element_type=jnp.float32))
        gb = (pb_ref[:, (k - 1 - t) * blk:(k - t) * blk]
              + lax.dot_general(ub, h_b, dm, preferred_element_type=jnp.float32))
        i_f = _sigmoid_pre(gf[0:HID])
        f_f = _sigmoid_pre(gf[HID:2 * HID])
        g_f = jnp.tanh(gf[2 * HID:3 * HID])
        o_f = _sigmoid_pre(gf[3 * HID:4 * HID])
        c_f = f_f * c_f + i_f * g_f
        h_f = o_f * jnp.tanh(c_f)
        i_b = _sigmoid_pre(gb[0:HID])
        f_b = _sigmoid_pre(gb[HID:2 * HID])
        g_b = jnp.tanh(gb[2 * HID:3 * HID])
        o_b = _sigmoid_pre(gb[3 * HID:4 * HID])
        c_b = f_b * c_b + i_b * g_b
        h_b = o_b * jnp.tanh(c_b)
        acc_f = acc_f + h_f
        acc_b = acc_b + h_b
    cat = jnp.concatenate([acc_f, acc_b], axis=0) * (1.0 / k)
    out_ref[...] = jnp.transpose(cat)


def _tc_bilstm_mean(g_tmajor, wf, uf, bf, wb, ub, bb, *, blk=256):
    k, n, d = g_tmajor.shape
    assert n % blk == 0
    nb = n // blk
    return pl.pallas_call(
        _lstm_body,
        grid=(nb,),
        in_specs=[
            pl.BlockSpec((k, blk, g_tmajor.shape[2]), lambda i: (0, i, 0)),
            pl.BlockSpec((G4, d), lambda i: (0, 0)),
            pl.BlockSpec((G4, HID), lambda i: (0, 0)),
            pl.BlockSpec((G4, 1), lambda i: (0, 0)),
            pl.BlockSpec((G4, d), lambda i: (0, 0)),
            pl.BlockSpec((G4, HID), lambda i: (0, 0)),
            pl.BlockSpec((G4, 1), lambda i: (0, 0)),
        ],
        out_specs=pl.BlockSpec((blk, 2 * HID), lambda i: (i, 0)),
        out_shape=jax.ShapeDtypeStruct((n, 2 * HID), jnp.float32),
        scratch_shapes=[
            pltpu.VMEM((G4, k * blk), jnp.float32),
            pltpu.VMEM((G4, k * blk), jnp.float32),
        ],
    )(g_tmajor, wf, uf, bf, wb, ub, bb)


def _gate_scale():
    return jnp.concatenate([
        jnp.full((2 * HID, 1), 0.5, jnp.float32),
        jnp.ones((HID, 1), jnp.float32),
        jnp.full((HID, 1), 0.5, jnp.float32),
    ], axis=0)


def _prep(wih, whh, bih, bhh, s):
    return wih * s, whh * s, (bih + bhh).reshape(G4, 1) * s


@jax.jit
def kernel(x_paper, x_author, idx_paper_to_author, idx_author_to_paper,
           p_wih_f, p_whh_f, p_bih_f, p_bhh_f, p_wih_b, p_whh_b, p_bih_b, p_bhh_b,
           a_wih_f, a_whh_f, a_bih_f, a_bhh_f, a_wih_b, a_whh_b, a_bih_b, a_bhh_b):
    n, k = idx_paper_to_author.shape
    d = x_paper.shape[1]
    blk = 256
    n_pad = (n + blk - 1) // blk * blk
    s = _gate_scale()

    def flat_idx(idx):
        idx_t = jnp.transpose(idx.astype(jnp.int32))
        idx_t = jnp.pad(idx_t, ((0, 0), (0, n_pad - n)))
        return idx_t.reshape(-1)

    pwf, puf, pbf = _prep(p_wih_f, p_whh_f, p_bih_f, p_bhh_f, s)
    pwb, pub, pbb = _prep(p_wih_b, p_whh_b, p_bih_b, p_bhh_b, s)
    awf, auf, abf = _prep(a_wih_f, a_whh_f, a_bih_f, a_bhh_f, s)
    awb, aub, abb = _prep(a_wih_b, a_whh_b, a_bih_b, a_bhh_b, s)

    g0 = _sc_gather(x_paper, flat_idx(idx_paper_to_author)).reshape(k, n_pad, d)
    out_author = _tc_bilstm_mean(g0, pwf, puf, pbf, pwb, pub, pbb, blk=blk)[:n]
    g1 = _sc_gather(x_author, flat_idx(idx_author_to_paper)).reshape(k, n_pad, d)
    out_paper = _tc_bilstm_mean(g1, awf, auf, abf, awb, aub, abb, blk=blk)[:n]
    return (out_author, out_paper)


# speedup vs baseline: 1.1313x; 1.1282x over previous
import functools

import jax
import jax.numpy as jnp
from jax import lax
from jax.experimental import pallas as pl
from jax.experimental.pallas import tpu as pltpu
from jax.experimental.pallas import tpu_sc as plsc

HID = 64
G4 = 4 * HID


def _sc_gather(table, idx_flat, *, chunk=320):
    V, D = table.shape
    R = idx_flat.shape[0]
    info = plsc.get_sparse_core_info()
    nw = info.num_cores * info.num_subcores
    assert R % nw == 0
    per_w = R // nw
    assert per_w % chunk == 0 and chunk % 8 == 0
    n_chunks = per_w // chunk
    mesh = plsc.VectorSubcoreMesh(core_axis_name="c", subcore_axis_name="s")

    @functools.partial(
        pl.kernel,
        mesh=mesh,
        out_type=jax.ShapeDtypeStruct((R, D), table.dtype),
        scratch_types=[
            pltpu.VMEM((chunk,), jnp.int32),
            pltpu.VMEM((chunk,), jnp.int32),
            pltpu.VMEM((chunk, D), table.dtype),
            pltpu.VMEM((chunk, D), table.dtype),
            pltpu.SemaphoreType.DMA,
            pltpu.SemaphoreType.DMA,
            pltpu.SemaphoreType.DMA,
            pltpu.SemaphoreType.DMA,
        ],
    )
    def gather_kernel(table_hbm, idx_hbm, out_hbm, idx0, idx1, rows0, rows1,
                      sg0, sg1, sw0, sw1):
        wid = lax.axis_index("s") * info.num_cores + lax.axis_index("c")
        base = wid * per_w
        idx_v = (idx0, idx1)
        rows_v = (rows0, rows1)
        sg = (sg0, sg1)
        sw = (sw0, sw1)

        def off(g):
            return base + g * chunk

        pltpu.sync_copy(idx_hbm.at[pl.ds(off(0), chunk)], idx_v[0])
        gathers = {0: pltpu.async_copy(table_hbm.at[idx_v[0]], rows_v[0], sg[0])}
        writes = {}
        for g in range(n_chunks):
            b = g & 1
            o = 1 - b
            if g + 1 < n_chunks:
                if g >= 1:
                    writes[g - 1].wait()
                pltpu.sync_copy(idx_hbm.at[pl.ds(off(g + 1), chunk)], idx_v[o])
                gathers[g + 1] = pltpu.async_copy(
                    table_hbm.at[idx_v[o]], rows_v[o], sg[o])
            gathers[g].wait()
            writes[g] = pltpu.async_copy(
                rows_v[b], out_hbm.at[pl.ds(off(g), chunk), :], sw[b])
        if n_chunks >= 2:
            writes[n_chunks - 2].wait()
        writes[n_chunks - 1].wait()

    return gather_kernel(table, idx_flat)


def _sigmoid_pre(y):
    return 0.5 * jnp.tanh(y) + 0.5


def _lstm_body(g_ref, wf_ref, uf_ref, bf_ref, wb_ref, ub_ref, bb_ref, out_ref,
               pf_ref, pb_ref):
    k, blk, d = g_ref.shape

    x_all = g_ref[...].reshape(k * blk, d)
    dn = (((1,), (1,)), ((), ()))
    wdt = x_all.dtype
    pf_ref[...] = (lax.dot_general(wf_ref[...].astype(wdt), x_all, dn,
                                   preferred_element_type=jnp.float32) + bf_ref[...])
    pb_ref[...] = (lax.dot_general(wb_ref[...].astype(wdt), x_all, dn,
                                   preferred_element_type=jnp.float32) + bb_ref[...])

    uf = uf_ref[...]
    ub = ub_ref[...]
    z = jnp.zeros((HID, blk), jnp.float32)
    h_f, c_f, h_b, c_b, acc_f, acc_b = z, z, z, z, z, z
    dm = (((1,), (0,)), ((), ()))
    for t in range(k):
        gf = (pf_ref[:, t * blk:(t + 1) * blk]
              + lax.dot_general(uf, h_f, dm, preferred_element_type=jnp.float32))
        gb = (pb_ref[:, (k - 1 - t) * blk:(k - t) * blk]
              + lax.dot_general(ub, h_b, dm, preferred_element_type=jnp.float32))
        i_f = _sigmoid_pre(gf[0:HID])
        f_f = _sigmoid_pre(gf[HID:2 * HID])
        g_f = jnp.tanh(gf[2 * HID:3 * HID])
        o_f = _sigmoid_pre(gf[3 * HID:4 * HID])
        c_f = f_f * c_f + i_f * g_f
        h_f = o_f * jnp.tanh(c_f)
        i_b = _sigmoid_pre(gb[0:HID])
        f_b = _sigmoid_pre(gb[HID:2 * HID])
        g_b = jnp.tanh(gb[2 * HID:3 * HID])
        o_b = _sigmoid_pre(gb[3 * HID:4 * HID])
        c_b = f_b * c_b + i_b * g_b
        h_b = o_b * jnp.tanh(c_b)
        acc_f = acc_f + h_f
        acc_b = acc_b + h_b
    cat = jnp.concatenate([acc_f, acc_b], axis=0) * (1.0 / k)
    out_ref[...] = jnp.transpose(cat)


def _tc_bilstm_mean(g_tmajor, wf, uf, bf, wb, ub, bb, *, blk=256):
    k, n, d = g_tmajor.shape
    assert n % blk == 0
    nb = n // blk
    return pl.pallas_call(
        _lstm_body,
        grid=(nb,),
        in_specs=[
            pl.BlockSpec((k, blk, g_tmajor.shape[2]), lambda i: (0, i, 0)),
            pl.BlockSpec((G4, d), lambda i: (0, 0)),
            pl.BlockSpec((G4, HID), lambda i: (0, 0)),
            pl.BlockSpec((G4, 1), lambda i: (0, 0)),
            pl.BlockSpec((G4, d), lambda i: (0, 0)),
            pl.BlockSpec((G4, HID), lambda i: (0, 0)),
            pl.BlockSpec((G4, 1), lambda i: (0, 0)),
        ],
        out_specs=pl.BlockSpec((blk, 2 * HID), lambda i: (i, 0)),
        out_shape=jax.ShapeDtypeStruct((n, 2 * HID), jnp.float32),
        scratch_shapes=[
            pltpu.VMEM((G4, k * blk), jnp.float32),
            pltpu.VMEM((G4, k * blk), jnp.float32),
        ],
    )(g_tmajor, wf, uf, bf, wb, ub, bb)


def _gate_scale():
    return jnp.concatenate([
        jnp.full((2 * HID, 1), 0.5, jnp.float32),
        jnp.ones((HID, 1), jnp.float32),
        jnp.full((HID, 1), 0.5, jnp.float32),
    ], axis=0)


def _prep(wih, whh, bih, bhh, s):
    return wih * s, whh * s, (bih + bhh).reshape(G4, 1) * s


@jax.jit
def kernel(x_paper, x_author, idx_paper_to_author, idx_author_to_paper,
           p_wih_f, p_whh_f, p_bih_f, p_bhh_f, p_wih_b, p_whh_b, p_bih_b, p_bhh_b,
           a_wih_f, a_whh_f, a_bih_f, a_bhh_f, a_wih_b, a_whh_b, a_bih_b, a_bhh_b):
    n, k = idx_paper_to_author.shape
    d = x_paper.shape[1]
    blk = 256
    half = n // 2
    h_pad = (half + blk - 1) // blk * blk
    s = _gate_scale()

    def flat_idx(idx, lo):
        idx_t = jnp.transpose(lax.slice_in_dim(idx, lo, lo + half, axis=0).astype(jnp.int32))
        idx_t = jnp.pad(idx_t, ((0, 0), (0, h_pad - half)))
        return idx_t.reshape(-1)

    pwf, puf, pbf = _prep(p_wih_f, p_whh_f, p_bih_f, p_bhh_f, s)
    pwb, pub, pbb = _prep(p_wih_b, p_whh_b, p_bih_b, p_bhh_b, s)
    awf, auf, abf = _prep(a_wih_f, a_whh_f, a_bih_f, a_bhh_f, s)
    awb, aub, abb = _prep(a_wih_b, a_whh_b, a_bih_b, a_bhh_b, s)

    outs = []
    for x_src, idx, wts in (
        (x_paper, idx_paper_to_author, (pwf, puf, pbf, pwb, pub, pbb)),
        (x_author, idx_author_to_paper, (awf, auf, abf, awb, aub, abb)),
    ):
        halves = []
        for lo in (0, half):
            g = _sc_gather(x_src, flat_idx(idx, lo)).reshape(k, h_pad, d)
            halves.append(_tc_bilstm_mean(g, *wts, blk=blk)[:half])
        outs.append(jnp.concatenate(halves, axis=0))
    return (outs[0], outs[1])
